# baseline (speedup 1.0000x reference)
import functools

import jax
import jax.numpy as jnp
from jax import lax
from jax.experimental import pallas as pl
from jax.experimental.pallas import tpu as pltpu
from jax.experimental.pallas import tpu_sc as plsc

_D = 128
_SCALE = float(_D) ** 0.5
_BATCH, _SEQ = 4096, 200

_R = 2000
_B1_BASE = 20000
_B2_BASE = 100000
_G2 = 58000
_TBL_ROWS = _B2_BASE + 16 * _G2
_S0, _S1, _S2 = 10, 40, 16 * (_G2 // _R)
_NSTEPS = _S0 + _S1 + _S2


def _table_body(w0, w1p, w2p, q, out):
    s = pl.program_id(0)
    dn = (((1,), (0,)), ((), ()))

    @pl.when(s < _S0)
    def _():
        out[...] = lax.dot_general(
            w0[...], q[0], dn, preferred_element_type=jnp.float32) * _SCALE

    @pl.when((s >= _S0) & (s < _S0 + _S1))
    def _():
        out[...] = lax.dot_general(
            w1p[...], q[0], dn, preferred_element_type=jnp.float32) * _SCALE

    @pl.when(s >= _S0 + _S1)
    def _():
        out[...] = lax.dot_general(
            w2p[...], q[0], dn, preferred_element_type=jnp.float32) * _SCALE


def _build_table(W0, W1, W2, P0, P1, P2):
    bf = jnp.bfloat16
    w0 = W0.astype(bf)
    w1p = W1.reshape(20000, 128).astype(bf)
    w2p = jnp.pad(W2.reshape(56250, 128), ((0, _G2 - 56250), (0, 0))).astype(bf)

    qs = [P0.T.astype(bf)]
    for j in range(4):
        qs.append(jnp.zeros((128, 128), bf).at[32 * j:32 * (j + 1), :]
                  .set(P1.T.astype(bf)))
    for j in range(16):
        qs.append(jnp.zeros((128, 128), bf).at[8 * j:8 * (j + 1), :]
                  .set(P2.T.astype(bf)))
    q = jnp.stack(qs)

    nb1 = _B1_BASE // _R
    nb2 = _G2 // _R

    return pl.pallas_call(
        _table_body,
        grid=(_NSTEPS,),
        in_specs=[
            pl.BlockSpec((_R, 128), lambda s: (jnp.where(s < _S0, s, 0), 0)),
            pl.BlockSpec(
                (_R, 128),
                lambda s: (jnp.where((s >= _S0) & (s < _S0 + _S1),
                                     (s - _S0) % nb1, 0), 0)),
            pl.BlockSpec(
                (_R, 128),
                lambda s: (jnp.where(s >= _S0 + _S1,
                                     (s - _S0 - _S1) % nb2, 0), 0)),
            pl.BlockSpec(
                (1, 128, 128),
                lambda s: (jnp.where(
                    s < _S0, 0,
                    jnp.where(s < _S0 + _S1,
                              1 + (s - _S0) // nb1,
                              5 + (s - _S0 - _S1) // nb2)), 0, 0)),
        ],
        out_specs=pl.BlockSpec((_R, 128), lambda s: (s, 0)),
        out_shape=jax.ShapeDtypeStruct((_TBL_ROWS, _D), jnp.float32),
    )(w0, w1p, w2p, q)


_N = _BATCH * _SEQ
_NC, _NS = 2, 16
_NW = _NC * _NS
_CH = 128
_NCH = _N // _NW // _CH

_sc_mesh = plsc.VectorSubcoreMesh(core_axis_name="c", subcore_axis_name="s")


@functools.partial(
    pl.kernel,
    mesh=_sc_mesh,
    out_type=jax.ShapeDtypeStruct((_N, _D), jnp.float32),
    scratch_types=[
        pltpu.VMEM((_NCH, _CH), jnp.int32),
        pltpu.VMEM((_CH, _D), jnp.float32),
        pltpu.SemaphoreType.DMA,
    ],
)
def _sc_gather(ids_hbm, table_hbm, out_hbm, idx_v, rows_v, sem):
    wid = lax.axis_index("s") * _NC + lax.axis_index("c")

    pltpu.sync_copy(ids_hbm.at[pl.ds(wid * _NCH, _NCH)], idx_v)

    def remap(c, carry):
        for k in range(_CH // 16):
            v = idx_v[c, pl.ds(k * 16, 16)]
            u1 = v - _B1_BASE
            p1 = _B1_BASE + (u1 & 3) * 20000 + (u1 >> 2)
            u2 = v - _B2_BASE
            p2 = _B2_BASE + (u2 & 15) * _G2 + (u2 >> 4)
            idx_v[c, pl.ds(k * 16, 16)] = jnp.where(
                v >= _B2_BASE, p2, jnp.where(v >= _B1_BASE, p1, v))
        return carry

    lax.fori_loop(0, _NCH, remap, 0)

    def body(c, carry):
        base = pl.multiple_of(wid * _NCH * _CH + c * _CH, _CH)
        pltpu.async_copy(table_hbm.at[idx_v.at[c]], rows_v, sem).wait()
        pltpu.sync_copy(rows_v, out_hbm.at[pl.ds(base, _CH)])
        return carry

    lax.fori_loop(0, _NCH, body, 0)


def kernel(input_, W0, W1, W2, P0, P1, P2):
    table = _build_table(W0, W1, W2, P0, P1, P2)
    ids = input_.reshape(_N // _CH, _CH)
    out = _sc_gather(ids, table)
    return out.reshape(_BATCH, _SEQ, _D)


# speedup vs baseline: 1.2967x; 1.2967x over previous
import functools

import jax
import jax.numpy as jnp
from jax import lax
from jax.experimental import pallas as pl
from jax.experimental.pallas import tpu as pltpu
from jax.experimental.pallas import tpu_sc as plsc

_D = 128
_SCALE = float(_D) ** 0.5
_BATCH, _SEQ = 4096, 200
_NUM_TOKENS = 1000000

_R = 10000
_NSTEPS = _NUM_TOKENS // _R
_S0 = 20000 // _R
_S1 = 100000 // _R


def _table_body(w0, q0, q1, q2, w1_hbm, w2_hbm, out,
                w1a, w1b, w2a, w2b, s1a, s1b, s2a, s2b):
    s = pl.program_id(0)
    dn = (((1,), (0,)), ((), ()))
    bf = jnp.bfloat16

    def w1_copy(t, buf, sem):
        return pltpu.make_async_copy(
            w1_hbm.at[pl.ds((t - _S0) * _R, _R), :], buf, sem)

    def w2_copy(t, buf, sem):
        return pltpu.make_async_copy(
            w2_hbm.at[pl.ds((t - _S1) * _R, _R), :], buf, sem)

    t = s + 1

    @pl.when((t >= _S0) & (t < _S1))
    def _():
        @pl.when(t % 2 == 0)
        def _():
            w1_copy(t, w1a, s1a).start()

        @pl.when(t % 2 == 1)
        def _():
            w1_copy(t, w1b, s1b).start()

    @pl.when((t >= _S1) & (t < _NSTEPS))
    def _():
        @pl.when(t % 2 == 0)
        def _():
            w2_copy(t, w2a, s2a).start()

        @pl.when(t % 2 == 1)
        def _():
            w2_copy(t, w2b, s2b).start()

    @pl.when(s < _S0)
    def _():
        out[...] = lax.dot_general(
            w0[...], q0[...], dn, preferred_element_type=jnp.float32) * _SCALE

    for par, (w1buf, w1sem, w2buf, w2sem) in enumerate(
            ((w1a, s1a, w2a, s2a), (w1b, s1b, w2b, s2b))):
        @pl.when((s >= _S0) & (s < _S1) & (s % 2 == par))
        def _(w1buf=w1buf, w1sem=w1sem):
            w1_copy(s, w1buf, w1sem).wait()
            out[...] = lax.dot_general(
                w1buf[...].astype(bf), q1[...], dn,
                preferred_element_type=jnp.float32) * _SCALE

        @pl.when((s >= _S1) & (s % 2 == par))
        def _(w2buf=w2buf, w2sem=w2sem):
            w2_copy(s, w2buf, w2sem).wait()
            out[...] = lax.dot_general(
                w2buf[...].astype(bf), q2[...], dn,
                preferred_element_type=jnp.float32) * _SCALE


def _build_table(W0, W1, W2, P0, P1, P2):
    bf = jnp.bfloat16
    return pl.pallas_call(
        _table_body,
        grid=(_NSTEPS,),
        in_specs=[
            pl.BlockSpec((_R, 128), lambda s: (jnp.where(s < _S0, s, 0), 0)),
            pl.BlockSpec((128, 128), lambda s: (0, 0)),
            pl.BlockSpec((32, 128), lambda s: (0, 0)),
            pl.BlockSpec((8, 128), lambda s: (0, 0)),
            pl.BlockSpec(memory_space=pl.ANY),
            pl.BlockSpec(memory_space=pl.ANY),
        ],
        out_specs=pl.BlockSpec((_R, _D), lambda s: (s, 0)),
        out_shape=jax.ShapeDtypeStruct((_NUM_TOKENS, _D), jnp.float32),
        scratch_shapes=[
            pltpu.VMEM((_R, 32), jnp.float32),
            pltpu.VMEM((_R, 32), jnp.float32),
            pltpu.VMEM((_R, 8), jnp.float32),
            pltpu.VMEM((_R, 8), jnp.float32),
            pltpu.SemaphoreType.DMA,
            pltpu.SemaphoreType.DMA,
            pltpu.SemaphoreType.DMA,
            pltpu.SemaphoreType.DMA,
        ],
    )(W0.astype(bf), P0.T.astype(bf), P1.T.astype(bf), P2.T.astype(bf),
      W1, W2)


_N = _BATCH * _SEQ
_NC, _NS = 2, 16
_NW = _NC * _NS
_PER_W = _N // _NW
_CH = 128
_NCH = _PER_W // _CH

_sc_mesh = plsc.VectorSubcoreMesh(core_axis_name="c", subcore_axis_name="s")


@functools.partial(
    pl.kernel,
    mesh=_sc_mesh,
    out_type=jax.ShapeDtypeStruct((_N, _D), jnp.float32),
    scratch_types=[
        pltpu.VMEM((_PER_W,), jnp.int32),
        pltpu.VMEM((_CH, _D), jnp.float32),
        pltpu.SemaphoreType.DMA,
    ],
)
def _sc_gather(ids_hbm, table_hbm, out_hbm, idx_v, rows_v, sem):
    wid = lax.axis_index("s") * _NC + lax.axis_index("c")

    pltpu.sync_copy(ids_hbm.at[pl.ds(wid * _PER_W, _PER_W)], idx_v)

    def body(c, carry):
        off = pl.multiple_of(c * _CH, _CH)
        pltpu.async_copy(
            table_hbm.at[idx_v.at[pl.ds(off, _CH)]], rows_v, sem).wait()
        pltpu.sync_copy(
            rows_v, out_hbm.at[pl.ds(pl.multiple_of(wid * _PER_W, _CH) + off,
                                     _CH)])
        return carry

    lax.fori_loop(0, _NCH, body, 0)


def kernel(input_, W0, W1, W2, P0, P1, P2):
    table = _build_table(W0, W1, W2, P0, P1, P2)
    ids = input_.reshape(_N)
    out = _sc_gather(ids, table)
    return out.reshape(_BATCH, _SEQ, _D)


# speedup vs baseline: 1.4647x; 1.1296x over previous
import functools

import jax
import jax.numpy as jnp
from jax import lax
from jax.experimental import pallas as pl
from jax.experimental.pallas import tpu as pltpu
from jax.experimental.pallas import tpu_sc as plsc

_D = 128
_SCALE = float(_D) ** 0.5
_BATCH, _SEQ = 4096, 200
_NUM_TOKENS = 1000000

_R = 10000
_NSTEPS = _NUM_TOKENS // _R
_S0 = 20000 // _R
_S1 = 100000 // _R


def _table_body(w0, w1, w2, q0, q1, q2, out):
    s = pl.program_id(0)
    dn = (((1,), (0,)), ((), ()))

    @pl.when(s < _S0)
    def _():
        out[...] = lax.dot_general(
            w0[...], q0[...], dn, preferred_element_type=jnp.float32)

    @pl.when((s >= _S0) & (s < _S1))
    def _():
        out[...] = lax.dot_general(
            w1[...], q1[...], dn, preferred_element_type=jnp.float32)

    @pl.when(s >= _S1)
    def _():
        out[...] = lax.dot_general(
            w2[...], q2[...], dn, preferred_element_type=jnp.float32)


def _build_table(W0, W1, W2, P0, P1, P2):
    bf = jnp.bfloat16
    return pl.pallas_call(
        _table_body,
        grid=(_NSTEPS,),
        in_specs=[
            pl.BlockSpec((_R, 128), lambda s: (jnp.where(s < _S0, s, 0), 0)),
            pl.BlockSpec((_R, 32),
                         lambda s: (jnp.clip(s - _S0, 0, _S1 - _S0 - 1), 0)),
            pl.BlockSpec((_R, 8),
                         lambda s: (jnp.clip(s - _S1, 0,
                                             _NSTEPS - _S1 - 1), 0)),
            pl.BlockSpec((128, 128), lambda s: (0, 0)),
            pl.BlockSpec((32, 128), lambda s: (0, 0)),
            pl.BlockSpec((8, 128), lambda s: (0, 0)),
        ],
        out_specs=pl.BlockSpec((_R, _D), lambda s: (s, 0)),
        out_shape=jax.ShapeDtypeStruct((_NUM_TOKENS, _D), jnp.float32),
    )(W0.astype(bf), W1.astype(bf), W2.astype(bf),
      (P0.T * _SCALE).astype(bf), (P1.T * _SCALE).astype(bf),
      (P2.T * _SCALE).astype(bf))


_N = _BATCH * _SEQ
_NC, _NS = 2, 16
_NW = _NC * _NS
_PER_W = _N // _NW
_CH = 128
_NCH = _PER_W // _CH

_sc_mesh = plsc.VectorSubcoreMesh(core_axis_name="c", subcore_axis_name="s")


@functools.partial(
    pl.kernel,
    mesh=_sc_mesh,
    out_type=jax.ShapeDtypeStruct((_N, _D), jnp.float32),
    scratch_types=[
        pltpu.VMEM((_PER_W,), jnp.int32),
        pltpu.VMEM((_CH, _D), jnp.float32),
        pltpu.SemaphoreType.DMA,
    ],
)
def _sc_gather(ids_hbm, table_hbm, out_hbm, idx_v, rows_v, sem):
    wid = lax.axis_index("s") * _NC + lax.axis_index("c")

    pltpu.sync_copy(ids_hbm.at[pl.ds(wid * _PER_W, _PER_W)], idx_v)

    def body(c, carry):
        off = pl.multiple_of(c * _CH, _CH)
        pltpu.async_copy(
            table_hbm.at[idx_v.at[pl.ds(off, _CH)]], rows_v, sem).wait()
        pltpu.sync_copy(
            rows_v, out_hbm.at[pl.ds(pl.multiple_of(wid * _PER_W, _CH) + off,
                                     _CH)])
        return carry

    lax.fori_loop(0, _NCH, body, 0)


def kernel(input_, W0, W1, W2, P0, P1, P2):
    table = _build_table(W0, W1, W2, P0, P1, P2)
    ids = input_.reshape(_N)
    out = _sc_gather(ids, table)
    return out.reshape(_BATCH, _SEQ, _D)
